# baseline (speedup 1.0000x reference)
import jax
import jax.numpy as jnp
from jax import lax
from jax.experimental import pallas as pl
from jax.experimental.pallas import tpu as pltpu
from jax.experimental.pallas import tpu_sc as plsc

DIM = 4096
BATCH = 16384
ROWS = 4

_INFO = plsc.get_sparse_core_info()
_NC = _INFO.num_cores
_NS = _INFO.num_subcores
_NW = _NC * _NS
_BPW = BATCH // _NW
_K = 16


def _body(ids_hbm, table_hbm, out_hbm, ids_v, table_v, dsem):
    wid = lax.axis_index("s") * _NC + lax.axis_index("c")
    base = wid * _BPW
    pltpu.sync_copy(ids_hbm.at[pl.ds(base, _BPW)], ids_v)
    pltpu.sync_copy(table_hbm, table_v)

    def issue(gbase):
        ids_vec = ids_v[pl.ds(gbase, _K)]
        for j in range(_K):
            r = ids_vec[j]
            pltpu.async_copy(table_v.at[r], out_hbm.at[base + gbase + j], dsem)

    def drain():
        for _ in range(_K):
            pltpu.make_async_copy(table_v.at[0], out_hbm.at[base], dsem).wait()

    issue(0)

    def group(g, carry):
        issue(g * _K)
        drain()
        return carry

    lax.fori_loop(1, _BPW // _K, group, 0)
    drain()


def kernel(step_ids, step_embeddings):
    ids = step_ids.astype(jnp.int32)
    out = pl.kernel(
        _body,
        out_type=jax.ShapeDtypeStruct((BATCH, DIM), jnp.float32),
        mesh=plsc.VectorSubcoreMesh(core_axis_name="c", subcore_axis_name="s"),
        scratch_types=[
            pltpu.VMEM((_BPW,), jnp.int32),
            pltpu.VMEM((ROWS, DIM), jnp.float32),
            pltpu.SemaphoreType.DMA,
        ],
    )(ids, step_embeddings)
    return out[:, None, :]


# speedup vs baseline: 1.8704x; 1.8704x over previous
import jax
import jax.numpy as jnp
from jax import lax
from jax.experimental import pallas as pl
from jax.experimental.pallas import tpu as pltpu
from jax.experimental.pallas import tpu_sc as plsc

DIM = 4096
BATCH = 16384
ROWS = 4

_INFO = plsc.get_sparse_core_info()
_NC = _INFO.num_cores
_NS = _INFO.num_subcores
_NW = _NC * _NS
_BPW = BATCH // _NW
_NTRI = 160
_NPAIR = 16
_G = 16


def _body(ids_hbm, table_hbm, out_hbm, ids_v, trip_s, bsem, dsem, psem):
    cid = lax.axis_index("c")
    sid = lax.axis_index("s")
    wid = sid * _NC + cid
    base = wid * _BPW
    pltpu.sync_copy(ids_hbm.at[pl.ds(base, _BPW)], ids_v)

    for k in range(4):
        c = sid * 4 + k
        r0 = c // 16
        r1 = (c // 4) % 4
        r2 = c % 4
        pltpu.async_copy(table_hbm.at[pl.ds(r0 * DIM, DIM)], trip_s.at[pl.ds(c * 3 * DIM, DIM)], bsem)
        pltpu.async_copy(table_hbm.at[pl.ds(r1 * DIM, DIM)], trip_s.at[pl.ds((c * 3 + 1) * DIM, DIM)], bsem)
        pltpu.async_copy(table_hbm.at[pl.ds(r2 * DIM, DIM)], trip_s.at[pl.ds((c * 3 + 2) * DIM, DIM)], bsem)
    for _ in range(12):
        pltpu.make_async_copy(table_hbm.at[pl.ds(0, DIM)], trip_s.at[pl.ds(0, DIM)], bsem).wait()
    plsc.subcore_barrier()

    def issue_triples(tb):
        i0 = 3 * tb
        vs = (
            ids_v[pl.ds(i0, 16)],
            ids_v[pl.ds(i0 + 16, 16)],
            ids_v[pl.ds(i0 + 32, 16)],
        )
        for j in range(_G):
            e0 = vs[(3 * j) // 16][(3 * j) % 16]
            e1 = vs[(3 * j + 1) // 16][(3 * j + 1) % 16]
            e2 = vs[(3 * j + 2) // 16][(3 * j + 2) % 16]
            combo = e0 * 16 + e1 * 4 + e2
            pltpu.async_copy(
                trip_s.at[pl.ds(combo * (3 * DIM), 3 * DIM)],
                out_hbm.at[pl.ds((base + 3 * (tb + j)) * DIM, 3 * DIM)],
                dsem,
            )

    def drain_triples():
        for _ in range(_G):
            pltpu.make_async_copy(trip_s.at[pl.ds(0, 3 * DIM)], out_hbm.at[pl.ds(0, 3 * DIM)], dsem).wait()

    issue_triples(0)

    def group(g, carry):
        issue_triples(g * _G)
        drain_triples()
        return carry

    lax.fori_loop(1, _NTRI // _G, group, 0)

    ws = (
        ids_v[pl.ds(3 * _NTRI, 16)],
        ids_v[pl.ds(3 * _NTRI + 16, 16)],
    )
    for j in range(_NPAIR):
        f0 = ws[(2 * j) // 16][(2 * j) % 16]
        f1 = ws[(2 * j + 1) // 16][(2 * j + 1) % 16]
        combo2 = f0 * 16 + f1 * 4
        pltpu.async_copy(
            trip_s.at[pl.ds(combo2 * (3 * DIM), 2 * DIM)],
            out_hbm.at[pl.ds((base + 3 * _NTRI + 2 * j) * DIM, 2 * DIM)],
            psem,
        )

    drain_triples()
    for _ in range(_NPAIR):
        pltpu.make_async_copy(trip_s.at[pl.ds(0, 2 * DIM)], out_hbm.at[pl.ds(0, 2 * DIM)], psem).wait()


def kernel(step_ids, step_embeddings):
    ids = step_ids.astype(jnp.int32)
    out = pl.kernel(
        _body,
        out_type=jax.ShapeDtypeStruct((BATCH * DIM,), jnp.float32),
        mesh=plsc.VectorSubcoreMesh(core_axis_name="c", subcore_axis_name="s"),
        scratch_types=[
            pltpu.VMEM((_BPW,), jnp.int32),
            pltpu.VMEM_SHARED((64 * 3 * DIM,), jnp.float32),
            pltpu.SemaphoreType.DMA,
            pltpu.SemaphoreType.DMA,
            pltpu.SemaphoreType.DMA,
        ],
    )(ids, step_embeddings.reshape(-1))
    return out.reshape(BATCH, 1, DIM)


# speedup vs baseline: 2.2247x; 1.1894x over previous
import jax
import jax.numpy as jnp
from jax import lax
from jax.experimental import pallas as pl
from jax.experimental.pallas import tpu as pltpu
from jax.experimental.pallas import tpu_sc as plsc

DIM = 4096
BATCH = 16384
ROWS = 4

_INFO = plsc.get_sparse_core_info()
_NC = _INFO.num_cores
_NS = _INFO.num_subcores
_NW = _NC * _NS
_BPW = BATCH // _NW
_NTRI = 112
_NSING = 176
_G = 16
_SCHED = ["T" if i in (0, 3, 5, 8, 10, 13, 15) else "S" for i in range(18)]


def _body(ids_hbm, table_hbm, out_hbm, ids_v, table_v, trip_s, bsem, dsem, ssem):
    cid = lax.axis_index("c")
    sid = lax.axis_index("s")
    wid = sid * _NC + cid
    base = wid * _BPW
    pltpu.sync_copy(ids_hbm.at[pl.ds(base, _BPW)], ids_v)
    pltpu.sync_copy(table_hbm, table_v)

    for k in range(4):
        c = sid * 4 + k
        r0 = c // 16
        r1 = (c // 4) % 4
        r2 = c % 4
        pltpu.async_copy(table_hbm.at[pl.ds(r0 * DIM, DIM)], trip_s.at[pl.ds(c * 3 * DIM, DIM)], bsem)
        pltpu.async_copy(table_hbm.at[pl.ds(r1 * DIM, DIM)], trip_s.at[pl.ds((c * 3 + 1) * DIM, DIM)], bsem)
        pltpu.async_copy(table_hbm.at[pl.ds(r2 * DIM, DIM)], trip_s.at[pl.ds((c * 3 + 2) * DIM, DIM)], bsem)
    for _ in range(12):
        pltpu.make_async_copy(table_hbm.at[pl.ds(0, DIM)], trip_s.at[pl.ds(0, DIM)], bsem).wait()
    plsc.subcore_barrier()

    def issue_triples(tb):
        i0 = 3 * tb
        vs = (
            ids_v[pl.ds(i0, 16)],
            ids_v[pl.ds(i0 + 16, 16)],
            ids_v[pl.ds(i0 + 32, 16)],
        )
        for j in range(_G):
            e0 = vs[(3 * j) // 16][(3 * j) % 16]
            e1 = vs[(3 * j + 1) // 16][(3 * j + 1) % 16]
            e2 = vs[(3 * j + 2) // 16][(3 * j + 2) % 16]
            combo = e0 * 16 + e1 * 4 + e2
            pltpu.async_copy(
                trip_s.at[pl.ds(combo * (3 * DIM), 3 * DIM)],
                out_hbm.at[pl.ds((base + 3 * (tb + j)) * DIM, 3 * DIM)],
                dsem,
            )

    def issue_singles(sb):
        v = ids_v[pl.ds(3 * _NTRI + sb, 16)]
        for j in range(_G):
            e = v[j]
            pltpu.async_copy(
                table_v.at[pl.ds(e * DIM, DIM)],
                out_hbm.at[pl.ds((base + 3 * _NTRI + sb + j) * DIM, DIM)],
                ssem,
            )

    def drain_triples():
        for _ in range(_G):
            pltpu.make_async_copy(trip_s.at[pl.ds(0, 3 * DIM)], out_hbm.at[pl.ds(0, 3 * DIM)], dsem).wait()

    def drain_singles():
        for _ in range(_G):
            pltpu.make_async_copy(table_v.at[pl.ds(0, DIM)], out_hbm.at[pl.ds(0, DIM)], ssem).wait()

    t_issued = 0
    s_issued = 0
    t_drained = 0
    s_drained = 0
    for typ in _SCHED:
        if typ == "T":
            if t_issued - t_drained >= 2:
                drain_triples()
                t_drained += 1
            issue_triples(t_issued * _G)
            t_issued += 1
        else:
            if s_issued - s_drained >= 2:
                drain_singles()
                s_drained += 1
            issue_singles(s_issued * _G)
            s_issued += 1
    while t_drained < t_issued:
        drain_triples()
        t_drained += 1
    while s_drained < s_issued:
        drain_singles()
        s_drained += 1


def kernel(step_ids, step_embeddings):
    ids = step_ids.astype(jnp.int32)
    out = pl.kernel(
        _body,
        out_type=jax.ShapeDtypeStruct((BATCH * DIM,), jnp.float32),
        mesh=plsc.VectorSubcoreMesh(core_axis_name="c", subcore_axis_name="s"),
        scratch_types=[
            pltpu.VMEM((_BPW,), jnp.int32),
            pltpu.VMEM((ROWS * DIM,), jnp.float32),
            pltpu.VMEM_SHARED((64 * 3 * DIM,), jnp.float32),
            pltpu.SemaphoreType.DMA,
            pltpu.SemaphoreType.DMA,
            pltpu.SemaphoreType.DMA,
        ],
    )(ids, step_embeddings.reshape(-1))
    return out.reshape(BATCH, 1, DIM)


# speedup vs baseline: 2.2362x; 1.0052x over previous
import jax
import jax.numpy as jnp
from jax import lax
from jax.experimental import pallas as pl
from jax.experimental.pallas import tpu as pltpu
from jax.experimental.pallas import tpu_sc as plsc

DIM = 4096
BATCH = 16384
ROWS = 4

_INFO = plsc.get_sparse_core_info()
_NC = _INFO.num_cores
_NS = _INFO.num_subcores
_NW = _NC * _NS
_BPW = BATCH // _NW
_NTRI = 112
_NSING = 176
_G = 16
_SCHED = ["T" if i in (0, 3, 5, 8, 10, 13, 15) else "S" for i in range(18)]


def _body(ids_hbm, table_hbm, out_hbm, ids_v, table_v, trip_s, bsem, dsem_a, dsem_b, ssem_a, ssem_b):
    cid = lax.axis_index("c")
    sid = lax.axis_index("s")
    wid = sid * _NC + cid
    base = wid * _BPW
    pltpu.sync_copy(ids_hbm.at[pl.ds(base, _BPW)], ids_v)
    pltpu.sync_copy(table_hbm, table_v)

    for k in range(4):
        c = sid * 4 + k
        r0 = c // 16
        r1 = (c // 4) % 4
        r2 = c % 4
        pltpu.async_copy(table_hbm.at[pl.ds(r0 * DIM, DIM)], trip_s.at[pl.ds(c * 3 * DIM, DIM)], bsem)
        pltpu.async_copy(table_hbm.at[pl.ds(r1 * DIM, DIM)], trip_s.at[pl.ds((c * 3 + 1) * DIM, DIM)], bsem)
        pltpu.async_copy(table_hbm.at[pl.ds(r2 * DIM, DIM)], trip_s.at[pl.ds((c * 3 + 2) * DIM, DIM)], bsem)
    for _ in range(12):
        pltpu.make_async_copy(table_hbm.at[pl.ds(0, DIM)], trip_s.at[pl.ds(0, DIM)], bsem).wait()
    plsc.subcore_barrier()

    dsems = (dsem_a, dsem_b)
    ssems = (ssem_a, ssem_b)

    def issue_triples(tb, sem):
        i0 = 3 * tb
        vs = (
            ids_v[pl.ds(i0, 16)],
            ids_v[pl.ds(i0 + 16, 16)],
            ids_v[pl.ds(i0 + 32, 16)],
        )
        for j in range(_G):
            e0 = vs[(3 * j) // 16][(3 * j) % 16]
            e1 = vs[(3 * j + 1) // 16][(3 * j + 1) % 16]
            e2 = vs[(3 * j + 2) // 16][(3 * j + 2) % 16]
            combo = e0 * 16 + e1 * 4 + e2
            pltpu.async_copy(
                trip_s.at[pl.ds(combo * (3 * DIM), 3 * DIM)],
                out_hbm.at[pl.ds((base + 3 * (tb + j)) * DIM, 3 * DIM)],
                sem,
            )

    def issue_singles(sb, sem):
        v = ids_v[pl.ds(3 * _NTRI + sb, 16)]
        for j in range(_G):
            e = v[j]
            pltpu.async_copy(
                table_v.at[pl.ds(e * DIM, DIM)],
                out_hbm.at[pl.ds((base + 3 * _NTRI + sb + j) * DIM, DIM)],
                sem,
            )

    def drain_triples(sem):
        for _ in range(_G):
            pltpu.make_async_copy(trip_s.at[pl.ds(0, 3 * DIM)], out_hbm.at[pl.ds(0, 3 * DIM)], sem).wait()

    def drain_singles(sem):
        for _ in range(_G):
            pltpu.make_async_copy(table_v.at[pl.ds(0, DIM)], out_hbm.at[pl.ds(0, DIM)], sem).wait()

    t_issued = 0
    s_issued = 0
    t_drained = [0, 0]
    s_drained = [0, 0]
    for typ in _SCHED:
        if typ == "T":
            q = t_issued % 2
            if t_issued // 2 - t_drained[q] >= 2:
                drain_triples(dsems[q])
                t_drained[q] += 1
            issue_triples(t_issued * _G, dsems[q])
            t_issued += 1
        else:
            q = s_issued % 2
            if s_issued // 2 - s_drained[q] >= 2:
                drain_singles(ssems[q])
                s_drained[q] += 1
            issue_singles(s_issued * _G, ssems[q])
            s_issued += 1
    for q in range(2):
        while t_drained[q] < (t_issued + 1 - q) // 2:
            drain_triples(dsems[q])
            t_drained[q] += 1
        while s_drained[q] < (s_issued + 1 - q) // 2:
            drain_singles(ssems[q])
            s_drained[q] += 1


def kernel(step_ids, step_embeddings):
    ids = step_ids.astype(jnp.int32)
    out = pl.kernel(
        _body,
        out_type=jax.ShapeDtypeStruct((BATCH * DIM,), jnp.float32),
        mesh=plsc.VectorSubcoreMesh(core_axis_name="c", subcore_axis_name="s"),
        scratch_types=[
            pltpu.VMEM((_BPW,), jnp.int32),
            pltpu.VMEM((ROWS * DIM,), jnp.float32),
            pltpu.VMEM_SHARED((64 * 3 * DIM,), jnp.float32),
            pltpu.SemaphoreType.DMA,
            pltpu.SemaphoreType.DMA,
            pltpu.SemaphoreType.DMA,
            pltpu.SemaphoreType.DMA,
            pltpu.SemaphoreType.DMA,
        ],
    )(ids, step_embeddings.reshape(-1))
    return out.reshape(BATCH, 1, DIM)
